# Initial kernel scaffold; baseline (speedup 1.0000x reference)
import jax
import jax.numpy as jnp
from jax.experimental import pallas as pl


def kernel(userIds, adGroupIds, userTable, adGroupTable, W1, b1, W2, b2):
    raise NotImplementedError("write your pallas kernel here")


# speedup vs baseline: 5.7937x; 5.7937x over previous
import functools

import jax
import jax.numpy as jnp
from jax import lax
from jax.experimental import pallas as pl
from jax.experimental.pallas import tpu as pltpu
from jax.experimental.pallas import tpu_sc as plsc

_NC = 2
_NS = 16
_NW = _NC * _NS

_BATCH = 16384
_D_EMB = 128
_B_PER_W = _BATCH // _NW


def _gather_body(u_tbl, a_tbl, uid, aid, u_out, a_out, idx_v, rows_v, sem):
    wid = lax.axis_index("s") * _NC + lax.axis_index("c")
    base = wid * _B_PER_W
    pltpu.sync_copy(uid.at[pl.ds(base, _B_PER_W)], idx_v)
    pltpu.async_copy(u_tbl.at[idx_v], rows_v, sem).wait()
    pltpu.sync_copy(rows_v, u_out.at[pl.ds(base, _B_PER_W)])
    pltpu.sync_copy(aid.at[pl.ds(base, _B_PER_W)], idx_v)
    pltpu.async_copy(a_tbl.at[idx_v], rows_v, sem).wait()
    pltpu.sync_copy(rows_v, a_out.at[pl.ds(base, _B_PER_W)])


_sc_gather = functools.partial(
    pl.kernel,
    out_type=(
        jax.ShapeDtypeStruct((_BATCH, _D_EMB), jnp.float32),
        jax.ShapeDtypeStruct((_BATCH, _D_EMB), jnp.float32),
    ),
    mesh=plsc.VectorSubcoreMesh(core_axis_name="c", subcore_axis_name="s"),
    scratch_types=[
        pltpu.VMEM((_B_PER_W,), jnp.int32),
        pltpu.VMEM((_B_PER_W, _D_EMB), jnp.float32),
        pltpu.SemaphoreType.DMA,
    ],
)(_gather_body)


def _mlp_body(xu_ref, xa_ref, w1u_ref, w1a_ref, b1_ref, w2_ref, b2_ref, o_ref):
    h = (
        jnp.dot(xu_ref[...], w1u_ref[...], preferred_element_type=jnp.float32)
        + jnp.dot(xa_ref[...], w1a_ref[...], preferred_element_type=jnp.float32)
        + b1_ref[...]
    )
    h = jnp.maximum(h, 0.0)
    o = jnp.sum(h * w2_ref[...], axis=1, keepdims=True) + b2_ref[...]
    o_ref[...] = jax.nn.sigmoid(o)


def _mlp(xu, xa, w1u, w1a, b1, w2row, b2, block_b=2048):
    nb = _BATCH // block_b
    return pl.pallas_call(
        _mlp_body,
        grid=(nb,),
        in_specs=[
            pl.BlockSpec((block_b, _D_EMB), lambda i: (i, 0)),
            pl.BlockSpec((block_b, _D_EMB), lambda i: (i, 0)),
            pl.BlockSpec((_D_EMB, 1024), lambda i: (0, 0)),
            pl.BlockSpec((_D_EMB, 1024), lambda i: (0, 0)),
            pl.BlockSpec((1, 1024), lambda i: (0, 0)),
            pl.BlockSpec((1, 1024), lambda i: (0, 0)),
            pl.BlockSpec((1, 1), lambda i: (0, 0)),
        ],
        out_specs=pl.BlockSpec((block_b, 1), lambda i: (i, 0)),
        out_shape=jax.ShapeDtypeStruct((_BATCH, 1), jnp.float32),
        compiler_params=pltpu.CompilerParams(
            dimension_semantics=("arbitrary",),
        ),
    )(xu, xa, w1u, w1a, b1, w2row, b2)


@jax.jit
def kernel(userIds, adGroupIds, userTable, adGroupTable, W1, b1, W2, b2):
    uid = userIds.reshape(_BATCH)
    aid = adGroupIds.reshape(_BATCH)
    xu, xa = _sc_gather(userTable, adGroupTable, uid, aid)
    w1u = W1[:_D_EMB]
    w1a = W1[_D_EMB:]
    b1r = b1.reshape(1, 1024)
    w2row = W2.reshape(1, 1024)
    b2r = b2.reshape(1, 1)
    return _mlp(xu, xa, w1u, w1a, b1r, w2row, b2r)
